# Initial kernel scaffold; baseline (speedup 1.0000x reference)
import jax
import jax.numpy as jnp
from jax.experimental import pallas as pl


def kernel(x, init_w, init_b, layer0_w, layer0_b, layer1_w, layer1_b, layer2_w, layer2_b, fc1_w, fc1_b, fc2_w, fc2_b):
    raise NotImplementedError("write your pallas kernel here")


# speedup vs baseline: 1.8299x; 1.8299x over previous
import functools

import jax
import jax.numpy as jnp
from jax.experimental import pallas as pl
from jax.experimental.pallas import tpu as pltpu

_BF16 = jnp.bfloat16
_VLIM = int(56 * 2**20)
_BB = 8


def _cparams(sem):
    return pltpu.CompilerParams(dimension_semantics=sem, vmem_limit_bytes=_VLIM)


def _init_kernel(a_ref, w_ref, b_ref, o_ref):
    a = a_ref[...].reshape(_BB * 1024, 32)
    w = w_ref[...]
    bias = b_ref[...]
    half = (_BB * 1024) // 2
    h1 = jnp.dot(a[:half], w, preferred_element_type=jnp.float32)
    h2 = jnp.dot(a[half:], w, preferred_element_type=jnp.float32)
    y1 = jnp.maximum(h1[:, :128] + h1[:, 128:] + bias, 0.0).astype(_BF16)
    y2 = jnp.maximum(h2[:, :128] + h2[:, 128:] + bias, 0.0).astype(_BF16)
    o_ref[0:_BB // 2] = y1.reshape(_BB // 2, 1024, 128)
    o_ref[_BB // 2:_BB] = y2.reshape(_BB // 2, 1024, 128)


def _init_call(a0, rhs, bias):
    n = a0.shape[0]
    return pl.pallas_call(
        _init_kernel,
        out_shape=jax.ShapeDtypeStruct((n, 1024, 128), _BF16),
        grid=(n // _BB,),
        in_specs=[
            pl.BlockSpec((_BB, 1024, 32), lambda b: (b, 0, 0)),
            pl.BlockSpec((32, 256), lambda b: (0, 0)),
            pl.BlockSpec((1, 128), lambda b: (0, 0)),
        ],
        out_specs=pl.BlockSpec((_BB, 1024, 128), lambda b: (b, 0, 0)),
        compiler_params=_cparams(("parallel",)),
    )(a0, rhs, bias)


def _down_kernel(x_ref, w_ref, b_ref, o_ref, lhs_ref, *, l_out, wh, cin, cout,
                 fold):
    for i in range(_BB):
        for t in range(9):
            dy, dx = t // 3, t % 3
            pln = (dy % 2) * 2 + (dx % 2)
            st = (dy // 2) * wh + dx // 2
            lhs_ref[i * l_out:(i + 1) * l_out, t * cin:(t + 1) * cin] = (
                x_ref[i, pln, st:st + l_out, :])
    bias = b_ref[...]
    half = (_BB * l_out) // 2
    for s in range(2):
        h = jnp.dot(lhs_ref[s * half:(s + 1) * half], w_ref[...],
                    preferred_element_type=jnp.float32)
        if fold:
            y = jnp.maximum(h[:, :cout] + h[:, cout:] + bias, 0.0)
        else:
            y = jnp.maximum(h + bias, 0.0)
        o_ref[s * (_BB // 2):(s + 1) * (_BB // 2)] = (
            y.astype(_BF16).reshape(_BB // 2, l_out, cout))


def _down_call(planes, rhs, bias, *, l_out, wh, cin, cout, fold):
    n, _, p_rows, _ = planes.shape
    body = functools.partial(_down_kernel, l_out=l_out, wh=wh, cin=cin,
                             cout=cout, fold=fold)
    return pl.pallas_call(
        body,
        out_shape=jax.ShapeDtypeStruct((n, l_out, cout), _BF16),
        grid=(n // _BB,),
        in_specs=[
            pl.BlockSpec((_BB, 4, p_rows, cin), lambda b: (b, 0, 0, 0)),
            pl.BlockSpec(rhs.shape, lambda b: (0, 0)),
            pl.BlockSpec((1, cout), lambda b: (0, 0)),
        ],
        out_specs=pl.BlockSpec((_BB, l_out, cout), lambda b: (b, 0, 0)),
        scratch_shapes=[pltpu.VMEM((_BB * l_out, 9 * cin), _BF16)],
        compiler_params=_cparams(("parallel",)),
    )(planes, rhs, bias)


def _up_kernel(x_ref, w_ref, b_ref, o_ref, lhs_ref):
    for i in range(_BB):
        for t in range(4):
            p, q = t // 2, t % 2
            lhs_ref[i * 96:i * 96 + 91, t * 256:(t + 1) * 256] = (
                x_ref[i, p * 10 + q:p * 10 + q + 91, :])
    bias = b_ref[...]
    for g in range(4):
        a, b = g // 2, g % 2
        h = jnp.dot(lhs_ref[...], w_ref[g],
                    preferred_element_type=jnp.float32)
        y = jnp.maximum(h + bias, 0.0).astype(_BF16)
        for i in range(_BB):
            base = i * 96 + a * 10 + b
            blk = y[base:base + 80].reshape(8, 10, 256)[:, :8, :]
            o_ref[i, :, a, :, b * 256:(b + 1) * 256] = blk


def _up_call(xf, w4, bias):
    n = xf.shape[0]
    return pl.pallas_call(
        _up_kernel,
        out_shape=jax.ShapeDtypeStruct((n, 8, 2, 8, 512), _BF16),
        grid=(n // _BB,),
        in_specs=[
            pl.BlockSpec((_BB, 110, 256), lambda b: (b, 0, 0)),
            pl.BlockSpec((4, 1024, 256), lambda b: (0, 0, 0)),
            pl.BlockSpec((1, 256), lambda b: (0, 0)),
        ],
        out_specs=pl.BlockSpec((_BB, 8, 2, 8, 512), lambda b: (b, 0, 0, 0, 0)),
        scratch_shapes=[pltpu.VMEM((_BB * 96, 1024), _BF16)],
        compiler_params=_cparams(("parallel",)),
    )(xf, w4, bias)


def _fc_kernel(a_ref, w1_ref, b1_ref, w2_ref, o_ref, acc_ref):
    i = pl.program_id(1)

    @pl.when(i == 0)
    def _():
        acc_ref[...] = jnp.zeros_like(acc_ref)

    a = a_ref[...]
    acc_ref[...] += (
        jnp.dot(a[:, :2048], w1_ref[0:2048], preferred_element_type=jnp.float32)
        + jnp.dot(a[:, 2048:], w1_ref[2048:4096],
                  preferred_element_type=jnp.float32))

    @pl.when(i == pl.num_programs(1) - 1)
    def _():
        h = jnp.maximum(acc_ref[...] + b1_ref[...], 0.0).astype(_BF16)
        o_ref[0] = jnp.dot(h, w2_ref[...], preferred_element_type=jnp.float32)


def _fc_call(a, w1, b1, w2):
    mp, k = a.shape
    tk = 4096
    return pl.pallas_call(
        _fc_kernel,
        out_shape=jax.ShapeDtypeStruct((2, mp, 128), jnp.float32),
        grid=(2, k // tk),
        in_specs=[
            pl.BlockSpec((mp, tk), lambda j, i: (0, i)),
            pl.BlockSpec((tk, 256), lambda j, i: (i, j)),
            pl.BlockSpec((1, 256), lambda j, i: (0, j)),
            pl.BlockSpec((256, 128), lambda j, i: (j, 0)),
        ],
        out_specs=pl.BlockSpec((1, mp, 128), lambda j, i: (j, 0, 0)),
        scratch_shapes=[pltpu.VMEM((mp, 256), jnp.float32)],
        compiler_params=_cparams(("parallel", "arbitrary")),
    )(a, w1, b1, w2)


def kernel(x, init_w, init_b, layer0_w, layer0_b, layer1_w, layer1_b,
           layer2_w, layer2_b, fc1_w, fc1_b, fc2_w, fc2_b):
    n = x.shape[0]

    xb = jnp.transpose(x, (0, 2, 3, 1)).astype(_BF16)
    xp = jnp.pad(xb, ((0, 0), (1, 1), (1, 1), (0, 0)))
    cols = [xp[:, dy:dy + 32, dx:dx + 32, :]
            for dy in range(3) for dx in range(3)]
    a0 = jnp.concatenate(cols, axis=-1).reshape(n, 1024, 27)
    a0 = jnp.pad(a0, ((0, 0), (0, 0), (0, 5)))
    w27 = init_w.reshape(9, 128, 128)[:, :3, :].reshape(27, 128)
    k27 = jnp.arange(27)[:, None]
    rhs0 = jnp.concatenate(
        [jnp.where(k27 < 15, w27, 0), jnp.where(k27 >= 15, w27, 0)], axis=1)
    rhs0 = jnp.pad(rhs0, ((0, 5), (0, 0)))
    y0 = _init_call(a0, rhs0, init_b).reshape(n, 32, 32, 128)

    xp0 = jnp.pad(y0, ((0, 0), (1, 3), (1, 1), (0, 0)))
    pl0 = jnp.stack([xp0[:, p::2, q::2, :] for p in (0, 1) for q in (0, 1)],
                    axis=1).reshape(n, 4, 306, 128)
    w0 = layer0_w[0]
    kk = jnp.arange(1152)[:, None]
    rhs1 = jnp.concatenate(
        [jnp.where(kk < 640, w0, 0), jnp.where(kk >= 640, w0, 0)], axis=1)
    y1 = _down_call(pl0, rhs1, layer0_b, l_out=272, wh=17, cin=128, cout=128,
                    fold=True)
    y1 = y1.reshape(n, 16, 17, 128)[:, :, :16, :]

    xp1 = jnp.pad(y1, ((0, 0), (1, 3), (1, 1), (0, 0)))
    pl1 = jnp.stack([xp1[:, p::2, q::2, :] for p in (0, 1) for q in (0, 1)],
                    axis=1).reshape(n, 4, 90, 128)
    y2 = _down_call(pl1, layer1_w[0], layer1_b, l_out=72, wh=9, cin=128,
                    cout=256, fold=False)
    y2 = y2.reshape(n, 8, 9, 256)[:, :, :8, :]

    xp2 = jnp.pad(y2, ((0, 0), (1, 2), (1, 1), (0, 0))).reshape(n, 110, 256)
    y3 = _up_call(xp2, layer2_w, layer2_b)

    a = y3.reshape(n, 65536)
    parts = _fc_call(a, fc1_w, fc1_b, fc2_w)
    out = parts[0] + parts[1] + fc2_b
    return out[:, :10]


# speedup vs baseline: 2.2963x; 1.2549x over previous
import jax
import jax.numpy as jnp
from jax.experimental import pallas as pl
from jax.experimental.pallas import tpu as pltpu

_BF16 = jnp.bfloat16
_VLIM = int(56 * 2**20)
_BB = 8
_P0R = 312


def _cparams(sem):
    return pltpu.CompilerParams(dimension_semantics=sem, vmem_limit_bytes=_VLIM)


def _conv_kernel(a_ref, m_ref, w0_ref, b0_ref, w1_ref, b1_ref, w2_ref, b2_ref,
                 w3_ref, b3_ref, o_ref, p0, l0h, y1p, p1, l1h, y2p, l2h):
    a = a_ref[...].reshape(_BB * 4 * _P0R, 32)
    b0 = b0_ref[...]
    half = (_BB * 4 * _P0R) // 2
    for s in range(2):
        h = jnp.dot(a[s * half:(s + 1) * half], w0_ref[...],
                    preferred_element_type=jnp.float32)
        y = jnp.maximum(h[:, :128] + h[:, 128:] + b0, 0.0).astype(_BF16)
        msk = m_ref[...]
        for im in range(_BB // 2):
            blk = y[im * 4 * _P0R:(im + 1) * 4 * _P0R] * msk
            p0[(s * 4 + im) * 4 * _P0R:(s * 4 + im + 1) * 4 * _P0R, :] = blk

    for i in range(_BB):
        for t in range(9):
            dy, dx = t // 3, t % 3
            pln = (dy % 2) * 2 + (dx % 2)
            st = (dy // 2) * 17 + dx // 2
            src = (i * 4 + pln) * _P0R + st
            l0h[i * 272:(i + 1) * 272, t * 128:(t + 1) * 128] = (
                p0[src:src + 272, :])
    b1 = b1_ref[...]
    y1p[...] = jnp.zeros_like(y1p)
    for s in range(2):
        h = jnp.dot(l0h[s * 1088:(s + 1) * 1088], w1_ref[...],
                    preferred_element_type=jnp.float32)
        y = jnp.maximum(h[:, :128] + h[:, 128:] + b1, 0.0)
        for im in range(_BB // 2):
            i = s * 4 + im
            for yo in range(16):
                y1p[i * 360 + (yo + 1) * 18 + 1:i * 360 + (yo + 1) * 18 + 17,
                    :] = y[im * 272 + yo * 17:im * 272 + yo * 17 + 16, :]

    for i in range(_BB):
        for pln in range(4):
            pp, q = pln // 2, pln % 2
            for ii in range(10):
                base = i * 360 + (2 * ii + pp) * 18 + q
                p1[i, pln, ii * 9:ii * 9 + 9, :] = (
                    y1p[pl.Slice(base, 9, 2), :].astype(_BF16))

    for i in range(_BB):
        for t in range(9):
            dy, dx = t // 3, t % 3
            pln = (dy % 2) * 2 + (dx % 2)
            st = (dy // 2) * 9 + dx // 2
            l1h[i * 72:(i + 1) * 72, t * 128:(t + 1) * 128] = (
                p1[i, pln, st:st + 72, :])
    b2 = b2_ref[...]
    y2p[...] = jnp.zeros_like(y2p)
    for s in range(2):
        h = jnp.dot(l1h[s * 288:(s + 1) * 288], w2_ref[...],
                    preferred_element_type=jnp.float32)
        y = jnp.maximum(h + b2, 0.0).astype(_BF16)
        for im in range(_BB // 2):
            i = s * 4 + im
            for yo in range(8):
                y2p[i, (yo + 1) * 10 + 1:(yo + 1) * 10 + 9, :] = (
                    y[im * 72 + yo * 9:im * 72 + yo * 9 + 8, :])

    for i in range(_BB):
        for t in range(4):
            p, q = t // 2, t % 2
            l2h[i * 96:i * 96 + 91, t * 256:(t + 1) * 256] = (
                y2p[i, p * 10 + q:p * 10 + q + 91, :])
    b3 = b3_ref[...]
    for g in range(4):
        ga, gb = g // 2, g % 2
        h = jnp.dot(l2h[...], w3_ref[g], preferred_element_type=jnp.float32)
        y = jnp.maximum(h + b3, 0.0).astype(_BF16)
        for i in range(_BB):
            base = i * 96 + ga * 10 + gb
            blk = y[base:base + 80].reshape(8, 10, 256)[:, :8, :]
            o_ref[i, :, ga, :, gb * 256:(gb + 1) * 256] = blk


def _conv_call(a0, mask, rhs0, b0, rhs1, b1, w1, b1c, w2, b2c):
    n = a0.shape[0]
    return pl.pallas_call(
        _conv_kernel,
        out_shape=jax.ShapeDtypeStruct((n, 8, 2, 8, 512), _BF16),
        grid=(n // _BB,),
        in_specs=[
            pl.BlockSpec((_BB, 4, _P0R, 32), lambda b: (b, 0, 0, 0)),
            pl.BlockSpec((4 * _P0R, 128), lambda b: (0, 0)),
            pl.BlockSpec((32, 256), lambda b: (0, 0)),
            pl.BlockSpec((1, 128), lambda b: (0, 0)),
            pl.BlockSpec((1152, 256), lambda b: (0, 0)),
            pl.BlockSpec((1, 128), lambda b: (0, 0)),
            pl.BlockSpec((1152, 256), lambda b: (0, 0)),
            pl.BlockSpec((1, 256), lambda b: (0, 0)),
            pl.BlockSpec((4, 1024, 256), lambda b: (0, 0, 0)),
            pl.BlockSpec((1, 256), lambda b: (0, 0)),
        ],
        out_specs=pl.BlockSpec((_BB, 8, 2, 8, 512), lambda b: (b, 0, 0, 0, 0)),
        scratch_shapes=[
            pltpu.VMEM((_BB * 4 * _P0R, 128), _BF16),
            pltpu.VMEM((_BB * 272, 1152), _BF16),
            pltpu.VMEM((_BB * 360, 128), jnp.float32),
            pltpu.VMEM((_BB, 4, 96, 128), _BF16),
            pltpu.VMEM((_BB * 72, 1152), _BF16),
            pltpu.VMEM((_BB, 110, 256), _BF16),
            pltpu.VMEM((_BB * 96, 1024), _BF16),
        ],
        compiler_params=_cparams(("parallel",)),
    )(a0, mask, rhs0, b0, rhs1, b1, w1, b1c, w2, b2c)


def _fc_kernel(a_ref, w1_ref, b1_ref, w2_ref, o_ref, acc_ref):
    i = pl.program_id(1)

    @pl.when(i == 0)
    def _():
        acc_ref[...] = jnp.zeros_like(acc_ref)

    a = a_ref[...]
    acc_ref[...] += (
        jnp.dot(a[:, :2048], w1_ref[0:2048], preferred_element_type=jnp.float32)
        + jnp.dot(a[:, 2048:], w1_ref[2048:4096],
                  preferred_element_type=jnp.float32))

    @pl.when(i == pl.num_programs(1) - 1)
    def _():
        h = jnp.maximum(acc_ref[...] + b1_ref[...], 0.0).astype(_BF16)
        o_ref[0] = jnp.dot(h, w2_ref[...], preferred_element_type=jnp.float32)


def _fc_call(a, w1, b1, w2):
    mp, k = a.shape
    tk = 4096
    return pl.pallas_call(
        _fc_kernel,
        out_shape=jax.ShapeDtypeStruct((2, mp, 128), jnp.float32),
        grid=(2, k // tk),
        in_specs=[
            pl.BlockSpec((mp, tk), lambda j, i: (0, i)),
            pl.BlockSpec((tk, 256), lambda j, i: (i, j)),
            pl.BlockSpec((1, 256), lambda j, i: (0, j)),
            pl.BlockSpec((256, 128), lambda j, i: (j, 0)),
        ],
        out_specs=pl.BlockSpec((1, mp, 128), lambda j, i: (j, 0, 0)),
        scratch_shapes=[pltpu.VMEM((mp, 256), jnp.float32)],
        compiler_params=_cparams(("parallel", "arbitrary")),
    )(a, w1, b1, w2)


def kernel(x, init_w, init_b, layer0_w, layer0_b, layer1_w, layer1_b,
           layer2_w, layer2_b, fc1_w, fc1_b, fc2_w, fc2_b):
    n = x.shape[0]

    xb = jnp.transpose(x, (0, 2, 3, 1)).astype(_BF16)
    xpad = jnp.pad(xb, ((0, 0), (2, 4), (2, 2), (0, 0)))
    planes = []
    for p in (0, 1):
        for q in (0, 1):
            taps = [xpad[:, p + dy:p + dy + 35:2, q + dx:q + dx + 33:2, :]
                    for dy in range(3) for dx in range(3)]
            pk = jnp.concatenate(taps, axis=-1).reshape(n, 306, 27)
            planes.append(jnp.pad(pk, ((0, 0), (0, _P0R - 306), (0, 5))))
    a0 = jnp.stack(planes, axis=1)

    r = jnp.arange(4 * _P0R)
    pln, rr = r // _P0R, r % _P0R
    pi, pj = rr // 17, rr % 17
    pp, qq = pln // 2, pln % 2
    ok = ((pi >= 1 - pp) & (pi <= 16 - pp) & (pj >= 1 - qq) & (pj <= 16 - qq)
          & (rr < 306))
    mask = jnp.broadcast_to(ok[:, None], (4 * _P0R, 128)).astype(_BF16)

    w27 = init_w.reshape(9, 128, 128)[:, :3, :].reshape(27, 128)
    k27 = jnp.arange(27)[:, None]
    rhs0 = jnp.concatenate(
        [jnp.where(k27 < 15, w27, 0), jnp.where(k27 >= 15, w27, 0)], axis=1)
    rhs0 = jnp.pad(rhs0, ((0, 5), (0, 0)))

    w0 = layer0_w[0]
    kk = jnp.arange(1152)[:, None]
    rhs1 = jnp.concatenate(
        [jnp.where(kk < 640, w0, 0), jnp.where(kk >= 640, w0, 0)], axis=1)

    y3 = _conv_call(a0, mask, rhs0, init_b, rhs1, layer0_b,
                    layer1_w[0], layer1_b, layer2_w, layer2_b)

    a = y3.reshape(n, 65536)
    parts = _fc_call(a, fc1_w, fc1_b, fc2_w)
    out = parts[0] + parts[1] + fc2_b
    return out[:, :10]


# speedup vs baseline: 2.6344x; 1.1473x over previous
import jax
import jax.numpy as jnp
from jax.experimental import pallas as pl
from jax.experimental.pallas import tpu as pltpu

_BF16 = jnp.bfloat16
_VLIM = int(56 * 2**20)
_BB = 8
_P0R = 312


def _cparams(sem):
    return pltpu.CompilerParams(dimension_semantics=sem, vmem_limit_bytes=_VLIM)


def _conv_kernel(a_ref, m_ref, w0_ref, b0_ref, w1_ref, b1_ref, w2_ref, b2_ref,
                 w3_ref, b3_ref, o_ref, p0, l0h, y1p, p1, l1h, y2p, l2h):
    a = a_ref[...].reshape(_BB * 4 * _P0R, 32)
    b0 = b0_ref[...]
    half = (_BB * 4 * _P0R) // 2
    for s in range(2):
        h = jnp.dot(a[s * half:(s + 1) * half], w0_ref[...],
                    preferred_element_type=jnp.float32)
        y = jnp.maximum(h[:, :128] + h[:, 128:] + b0, 0.0).astype(_BF16)
        msk = m_ref[...]
        for im in range(_BB // 2):
            blk = y[im * 4 * _P0R:(im + 1) * 4 * _P0R] * msk
            p0[(s * 4 + im) * 4 * _P0R:(s * 4 + im + 1) * 4 * _P0R, :] = blk

    for i in range(_BB):
        for t in range(9):
            dy, dx = t // 3, t % 3
            pln = (dy % 2) * 2 + (dx % 2)
            st = (dy // 2) * 17 + dx // 2
            src = (i * 4 + pln) * _P0R + st
            l0h[i * 272:(i + 1) * 272, t * 128:(t + 1) * 128] = (
                p0[src:src + 272, :])
    b1 = b1_ref[...]
    y1p[...] = jnp.zeros_like(y1p)
    for s in range(2):
        h = jnp.dot(l0h[s * 1088:(s + 1) * 1088], w1_ref[...],
                    preferred_element_type=jnp.float32)
        y = jnp.maximum(h[:, :128] + h[:, 128:] + b1, 0.0)
        for im in range(_BB // 2):
            i = s * 4 + im
            for yo in range(16):
                y1p[i * 360 + (yo + 1) * 18 + 1:i * 360 + (yo + 1) * 18 + 17,
                    :] = y[im * 272 + yo * 17:im * 272 + yo * 17 + 16, :]

    for i in range(_BB):
        for pln in range(4):
            pp, q = pln // 2, pln % 2
            for ii in range(10):
                base = i * 360 + (2 * ii + pp) * 18 + q
                p1[i, pln, ii * 9:ii * 9 + 9, :] = (
                    y1p[pl.Slice(base, 9, 2), :].astype(_BF16))

    for i in range(_BB):
        for t in range(9):
            dy, dx = t // 3, t % 3
            pln = (dy % 2) * 2 + (dx % 2)
            st = (dy // 2) * 9 + dx // 2
            l1h[i * 72:(i + 1) * 72, t * 128:(t + 1) * 128] = (
                p1[i, pln, st:st + 72, :])
    b2 = b2_ref[...]
    y2p[...] = jnp.zeros_like(y2p)
    for s in range(2):
        h = jnp.dot(l1h[s * 288:(s + 1) * 288], w2_ref[...],
                    preferred_element_type=jnp.float32)
        y = jnp.maximum(h + b2, 0.0).astype(_BF16)
        for im in range(_BB // 2):
            i = s * 4 + im
            for yo in range(8):
                y2p[i, (yo + 1) * 10 + 1:(yo + 1) * 10 + 9, :] = (
                    y[im * 72 + yo * 9:im * 72 + yo * 9 + 8, :])

    for i in range(_BB):
        for t in range(4):
            p, q = t // 2, t % 2
            l2h[i * 96:i * 96 + 91, t * 256:(t + 1) * 256] = (
                y2p[i, p * 10 + q:p * 10 + q + 91, :])
    b3 = b3_ref[...]
    for g in range(4):
        ga, gb = g // 2, g % 2
        h = jnp.dot(l2h[...], w3_ref[g], preferred_element_type=jnp.float32)
        y = jnp.maximum(h + b3, 0.0).astype(_BF16)
        for i in range(_BB):
            base = i * 96 + ga * 10 + gb
            blk = y[base:base + 80].reshape(8, 10, 256)[:, :8, :]
            o_ref[i, :, ga, :, gb * 256:(gb + 1) * 256] = blk


def _conv_call(a0, mask, rhs0, b0, rhs1, b1, w1, b1c, w2, b2c):
    n = a0.shape[0]
    return pl.pallas_call(
        _conv_kernel,
        out_shape=jax.ShapeDtypeStruct((n, 8, 2, 8, 512), _BF16),
        grid=(n // _BB,),
        in_specs=[
            pl.BlockSpec((_BB, 4, _P0R, 32), lambda b: (b, 0, 0, 0)),
            pl.BlockSpec((4 * _P0R, 128), lambda b: (0, 0)),
            pl.BlockSpec((32, 256), lambda b: (0, 0)),
            pl.BlockSpec((1, 128), lambda b: (0, 0)),
            pl.BlockSpec((1152, 256), lambda b: (0, 0)),
            pl.BlockSpec((1, 128), lambda b: (0, 0)),
            pl.BlockSpec((1152, 256), lambda b: (0, 0)),
            pl.BlockSpec((1, 256), lambda b: (0, 0)),
            pl.BlockSpec((4, 1024, 256), lambda b: (0, 0, 0)),
            pl.BlockSpec((1, 256), lambda b: (0, 0)),
        ],
        out_specs=pl.BlockSpec((_BB, 8, 2, 8, 512), lambda b: (b, 0, 0, 0, 0)),
        scratch_shapes=[
            pltpu.VMEM((_BB * 4 * _P0R, 128), _BF16),
            pltpu.VMEM((_BB * 272, 1152), _BF16),
            pltpu.VMEM((_BB * 360, 128), jnp.float32),
            pltpu.VMEM((_BB, 4, 96, 128), _BF16),
            pltpu.VMEM((_BB * 72, 1152), _BF16),
            pltpu.VMEM((_BB, 110, 256), _BF16),
            pltpu.VMEM((_BB * 96, 1024), _BF16),
        ],
        compiler_params=_cparams(("parallel",)),
    )(a0, mask, rhs0, b0, rhs1, b1, w1, b1c, w2, b2c)


def _fc_kernel(a_ref, w1_ref, b1_ref, w2_ref, o_ref, acc_ref):
    i = pl.program_id(1)

    @pl.when(i == 0)
    def _():
        acc_ref[...] = jnp.zeros_like(acc_ref)

    a = a_ref[...]
    acc_ref[...] += (
        jnp.dot(a[:, :2048], w1_ref[0:2048], preferred_element_type=jnp.float32)
        + jnp.dot(a[:, 2048:], w1_ref[2048:4096],
                  preferred_element_type=jnp.float32))

    @pl.when(i == pl.num_programs(1) - 1)
    def _():
        h = jnp.maximum(acc_ref[...] + b1_ref[...], 0.0).astype(_BF16)
        o_ref[0] = jnp.dot(h, w2_ref[...], preferred_element_type=jnp.float32)


def _fc_call(a, w1, b1, w2):
    mp, k = a.shape
    tk = 4096
    return pl.pallas_call(
        _fc_kernel,
        out_shape=jax.ShapeDtypeStruct((2, mp, 128), jnp.float32),
        grid=(2, k // tk),
        in_specs=[
            pl.BlockSpec((mp, tk), lambda j, i: (0, i)),
            pl.BlockSpec((tk, 256), lambda j, i: (i, j)),
            pl.BlockSpec((1, 256), lambda j, i: (0, j)),
            pl.BlockSpec((256, 128), lambda j, i: (j, 0)),
        ],
        out_specs=pl.BlockSpec((1, mp, 128), lambda j, i: (j, 0, 0)),
        scratch_shapes=[pltpu.VMEM((mp, 256), jnp.float32)],
        compiler_params=_cparams(("parallel", "arbitrary")),
    )(a, w1, b1, w2)


def kernel(x, init_w, init_b, layer0_w, layer0_b, layer1_w, layer1_b,
           layer2_w, layer2_b, fc1_w, fc1_b, fc2_w, fc2_b):
    n = x.shape[0]

    xb = jnp.transpose(x, (0, 2, 3, 1)).astype(_BF16)
    xpad = jnp.pad(xb, ((0, 0), (2, 4), (2, 2), (0, 0)))
    taps = [xpad[:, dy:dy + 36, dx:dx + 34, :]
            for dy in range(3) for dx in range(3)]
    pd = jnp.concatenate(taps, axis=-1)
    pd = jnp.pad(pd, ((0, 0), (0, 0), (0, 0), (0, 5)))
    pd = pd.reshape(n, 18, 2, 17, 64)
    planes = []
    for p in (0, 1):
        for q in (0, 1):
            pk = pd[:, :, p, :, q * 32:(q + 1) * 32].reshape(n, 306, 32)
            planes.append(jnp.pad(pk, ((0, 0), (0, _P0R - 306), (0, 0))))
    a0 = jnp.stack(planes, axis=1)

    r = jnp.arange(4 * _P0R)
    pln, rr = r // _P0R, r % _P0R
    pi, pj = rr // 17, rr % 17
    pp, qq = pln // 2, pln % 2
    ok = ((pi >= 1 - pp) & (pi <= 16 - pp) & (pj >= 1 - qq) & (pj <= 16 - qq)
          & (rr < 306))
    mask = jnp.broadcast_to(ok[:, None], (4 * _P0R, 128)).astype(_BF16)

    w27 = init_w.reshape(9, 128, 128)[:, :3, :].reshape(27, 128)
    k27 = jnp.arange(27)[:, None]
    rhs0 = jnp.concatenate(
        [jnp.where(k27 < 15, w27, 0), jnp.where(k27 >= 15, w27, 0)], axis=1)
    rhs0 = jnp.pad(rhs0, ((0, 5), (0, 0)))

    w0 = layer0_w[0]
    kk = jnp.arange(1152)[:, None]
    rhs1 = jnp.concatenate(
        [jnp.where(kk < 640, w0, 0), jnp.where(kk >= 640, w0, 0)], axis=1)

    y3 = _conv_call(a0, mask, rhs0, init_b, rhs1, layer0_b,
                    layer1_w[0], layer1_b, layer2_w, layer2_b)

    a = y3.reshape(n, 65536)
    parts = _fc_call(a, fc1_w, fc1_b, fc2_w)
    out = parts[0] + parts[1] + fc2_b
    return out[:, :10]


# speedup vs baseline: 2.8911x; 1.0974x over previous
import jax
import jax.numpy as jnp
from jax.experimental import pallas as pl
from jax.experimental.pallas import tpu as pltpu

_BF16 = jnp.bfloat16
_VLIM = int(56 * 2**20)
_BB = 8
_P0R = 312


def _cparams(sem):
    return pltpu.CompilerParams(dimension_semantics=sem, vmem_limit_bytes=_VLIM)


def _conv_kernel(a_ref, m_ref, w0_ref, b0_ref, w1_ref, b1_ref, w2_ref, b2_ref,
                 w3_ref, b3_ref, o_ref, p0, l0h, y1p, p1, l1h, y2p, l2h):
    a = a_ref[...].reshape(_BB * 2 * _P0R, 64)
    b0 = b0_ref[...]
    msk = m_ref[...]
    half = (_BB * 2 * _P0R) // 2
    for s in range(2):
        h = jnp.dot(a[s * half:(s + 1) * half], w0_ref[...],
                    preferred_element_type=jnp.float32)
        for q in range(2):
            y = jnp.maximum(h[:, q * 256:q * 256 + 128]
                            + h[:, q * 256 + 128:(q + 1) * 256] + b0,
                            0.0).astype(_BF16)
            for im in range(_BB // 2):
                for p in range(2):
                    pln = p * 2 + q
                    dst = ((s * 4 + im) * 4 + pln) * _P0R
                    p0[dst:dst + _P0R, :] = (
                        y[(im * 2 + p) * _P0R:(im * 2 + p + 1) * _P0R, :]
                        * msk[pln * _P0R:(pln + 1) * _P0R, :])

    for i in range(_BB):
        for t in range(9):
            dy, dx = t // 3, t % 3
            pln = (dy % 2) * 2 + (dx % 2)
            st = (dy // 2) * 17 + dx // 2
            src = (i * 4 + pln) * _P0R + st
            l0h[i * 272:(i + 1) * 272, t * 128:(t + 1) * 128] = (
                p0[src:src + 272, :])
    b1 = b1_ref[...]
    y1p[...] = jnp.zeros_like(y1p)
    for s in range(2):
        h = jnp.dot(l0h[s * 1088:(s + 1) * 1088], w1_ref[...],
                    preferred_element_type=jnp.float32)
        y = jnp.maximum(h[:, :128] + h[:, 128:] + b1, 0.0)
        for im in range(_BB // 2):
            i = s * 4 + im
            for yo in range(16):
                y1p[i * 360 + (yo + 1) * 18 + 1:i * 360 + (yo + 1) * 18 + 17,
                    :] = y[im * 272 + yo * 17:im * 272 + yo * 17 + 16, :]

    for i in range(_BB):
        for pln in range(4):
            pp, q = pln // 2, pln % 2
            for ii in range(10):
                base = i * 360 + (2 * ii + pp) * 18 + q
                p1[i, pln, ii * 9:ii * 9 + 9, :] = (
                    y1p[pl.Slice(base, 9, 2), :].astype(_BF16))

    for i in range(_BB):
        for t in range(9):
            dy, dx = t // 3, t % 3
            pln = (dy % 2) * 2 + (dx % 2)
            st = (dy // 2) * 9 + dx // 2
            l1h[i * 72:(i + 1) * 72, t * 128:(t + 1) * 128] = (
                p1[i, pln, st:st + 72, :])
    b2 = b2_ref[...]
    y2p[...] = jnp.zeros_like(y2p)
    for s in range(2):
        h = jnp.dot(l1h[s * 288:(s + 1) * 288], w2_ref[...],
                    preferred_element_type=jnp.float32)
        y = jnp.maximum(h + b2, 0.0).astype(_BF16)
        for im in range(_BB // 2):
            i = s * 4 + im
            for yo in range(8):
                y2p[i, (yo + 1) * 10 + 1:(yo + 1) * 10 + 9, :] = (
                    y[im * 72 + yo * 9:im * 72 + yo * 9 + 8, :])

    for i in range(_BB):
        for t in range(4):
            p, q = t // 2, t % 2
            l2h[i * 96:i * 96 + 91, t * 256:(t + 1) * 256] = (
                y2p[i, p * 10 + q:p * 10 + q + 91, :])
    b3 = b3_ref[...]
    for g in range(4):
        ga, gb = g // 2, g % 2
        h = jnp.dot(l2h[...], w3_ref[g], preferred_element_type=jnp.float32)
        y = jnp.maximum(h + b3, 0.0).astype(_BF16)
        for i in range(_BB):
            base = i * 96 + ga * 10 + gb
            blk = y[base:base + 80].reshape(8, 10, 256)[:, :8, :]
            o_ref[i, :, ga, :, gb * 256:(gb + 1) * 256] = blk


def _conv_call(a0, mask, rhs0, b0, rhs1, b1, w1, b1c, w2, b2c):
    n = a0.shape[0]
    return pl.pallas_call(
        _conv_kernel,
        out_shape=jax.ShapeDtypeStruct((n, 8, 2, 8, 512), _BF16),
        grid=(n // _BB,),
        in_specs=[
            pl.BlockSpec((_BB, 2, _P0R, 64), lambda b: (b, 0, 0, 0)),
            pl.BlockSpec((4 * _P0R, 128), lambda b: (0, 0)),
            pl.BlockSpec((64, 512), lambda b: (0, 0)),
            pl.BlockSpec((1, 128), lambda b: (0, 0)),
            pl.BlockSpec((1152, 256), lambda b: (0, 0)),
            pl.BlockSpec((1, 128), lambda b: (0, 0)),
            pl.BlockSpec((1152, 256), lambda b: (0, 0)),
            pl.BlockSpec((1, 256), lambda b: (0, 0)),
            pl.BlockSpec((4, 1024, 256), lambda b: (0, 0, 0)),
            pl.BlockSpec((1, 256), lambda b: (0, 0)),
        ],
        out_specs=pl.BlockSpec((_BB, 8, 2, 8, 512), lambda b: (b, 0, 0, 0, 0)),
        scratch_shapes=[
            pltpu.VMEM((_BB * 4 * _P0R, 128), _BF16),
            pltpu.VMEM((_BB * 272, 1152), _BF16),
            pltpu.VMEM((_BB * 360, 128), jnp.float32),
            pltpu.VMEM((_BB, 4, 96, 128), _BF16),
            pltpu.VMEM((_BB * 72, 1152), _BF16),
            pltpu.VMEM((_BB, 110, 256), _BF16),
            pltpu.VMEM((_BB * 96, 1024), _BF16),
        ],
        compiler_params=_cparams(("parallel",)),
    )(a0, mask, rhs0, b0, rhs1, b1, w1, b1c, w2, b2c)


def _fc_kernel(a_ref, w1_ref, b1_ref, w2_ref, o_ref, acc_ref):
    i = pl.program_id(1)

    @pl.when(i == 0)
    def _():
        acc_ref[...] = jnp.zeros_like(acc_ref)

    a = a_ref[...]
    acc_ref[...] += (
        jnp.dot(a[:, :2048], w1_ref[0:2048], preferred_element_type=jnp.float32)
        + jnp.dot(a[:, 2048:], w1_ref[2048:4096],
                  preferred_element_type=jnp.float32))

    @pl.when(i == pl.num_programs(1) - 1)
    def _():
        h = jnp.maximum(acc_ref[...] + b1_ref[...], 0.0).astype(_BF16)
        o_ref[0] = jnp.dot(h, w2_ref[...], preferred_element_type=jnp.float32)


def _fc_call(a, w1, b1, w2):
    mp, k = a.shape
    tk = 4096
    return pl.pallas_call(
        _fc_kernel,
        out_shape=jax.ShapeDtypeStruct((2, mp, 128), jnp.float32),
        grid=(2, k // tk),
        in_specs=[
            pl.BlockSpec((mp, tk), lambda j, i: (0, i)),
            pl.BlockSpec((tk, 256), lambda j, i: (i, j)),
            pl.BlockSpec((1, 256), lambda j, i: (0, j)),
            pl.BlockSpec((256, 128), lambda j, i: (j, 0)),
        ],
        out_specs=pl.BlockSpec((1, mp, 128), lambda j, i: (j, 0, 0)),
        scratch_shapes=[pltpu.VMEM((mp, 256), jnp.float32)],
        compiler_params=_cparams(("parallel", "arbitrary")),
    )(a, w1, b1, w2)


def kernel(x, init_w, init_b, layer0_w, layer0_b, layer1_w, layer1_b,
           layer2_w, layer2_b, fc1_w, fc1_b, fc2_w, fc2_b):
    n = x.shape[0]

    xb = jnp.transpose(x, (0, 2, 3, 1)).astype(_BF16)
    xpad = jnp.pad(xb, ((0, 0), (2, 4), (2, 2), (0, 0)))
    taps = [xpad[:, dy:dy + 36, dx:dx + 34, :]
            for dy in range(3) for dx in range(3)]
    pd = jnp.concatenate(taps, axis=-1)
    pd = jnp.pad(pd, ((0, 0), (0, 0), (0, 0), (0, 5)))
    pd = pd.reshape(n, 18, 2, 17 * 64)
    a0 = jnp.stack([pd[:, :, 0, :], pd[:, :, 1, :]], axis=1)
    a0 = a0.reshape(n, 2, 306, 64)
    a0 = jnp.pad(a0, ((0, 0), (0, 0), (0, _P0R - 306), (0, 0)))

    r = jnp.arange(4 * _P0R)
    pln, rr = r // _P0R, r % _P0R
    pi, pj = rr // 17, rr % 17
    pp, qq = pln // 2, pln % 2
    ok = ((pi >= 1 - pp) & (pi <= 16 - pp) & (pj >= 1 - qq) & (pj <= 16 - qq)
          & (rr < 306))
    mask = jnp.broadcast_to(ok[:, None], (4 * _P0R, 128)).astype(_BF16)

    w27 = init_w.reshape(9, 128, 128)[:, :3, :].reshape(27, 128)
    k27 = jnp.arange(27)[:, None]
    rhs0 = jnp.concatenate(
        [jnp.where(k27 < 15, w27, 0), jnp.where(k27 >= 15, w27, 0)], axis=1)
    rhs0 = jnp.pad(rhs0, ((0, 5), (0, 0)))
    z = jnp.zeros_like(rhs0)
    rhs0 = jnp.concatenate([jnp.concatenate([rhs0, z], axis=1),
                            jnp.concatenate([z, rhs0], axis=1)], axis=0)

    w0 = layer0_w[0]
    kk = jnp.arange(1152)[:, None]
    rhs1 = jnp.concatenate(
        [jnp.where(kk < 640, w0, 0), jnp.where(kk >= 640, w0, 0)], axis=1)

    y3 = _conv_call(a0, mask, rhs0, init_b, rhs1, layer0_b,
                    layer1_w[0], layer1_b, layer2_w, layer2_b)

    a = y3.reshape(n, 65536)
    parts = _fc_call(a, fc1_w, fc1_b, fc2_w)
    out = parts[0] + parts[1] + fc2_b
    return out[:, :10]


# speedup vs baseline: 3.2231x; 1.1149x over previous
import jax
import jax.numpy as jnp
from jax.experimental import pallas as pl
from jax.experimental.pallas import tpu as pltpu

_BF16 = jnp.bfloat16
_VLIM = int(56 * 2**20)
_BB = 8
_P0R = 312


def _cparams(sem):
    return pltpu.CompilerParams(dimension_semantics=sem, vmem_limit_bytes=_VLIM)


def _conv_kernel(a_ref, m_ref, w0_ref, b0_ref, w1_ref, b1_ref, w2_ref, b2_ref,
                 w3_ref, b3_ref, o_ref, lhsT, p0d, p0, l0h, y1p, p1, l1h, y2p,
                 l2h):
    b0 = b0_ref[...]
    msk = m_ref[...]
    lhsT[27:32, :] = jnp.zeros((5, 1280), _BF16)
    for i in range(_BB):
        for t in range(9):
            dy, dx = t // 3, t % 3
            sh = dy * 36 + dx
            lhsT[t * 3:(t + 1) * 3, :] = a_ref[i, :, sh:sh + 1280]
        h = jnp.dot(lhsT[...].T, w0_ref[...],
                    preferred_element_type=jnp.float32)
        y = jnp.maximum(h[:, :128] + h[:, 128:] + b0, 0.0)
        p0d[i * 1280:(i + 1) * 1280, :] = y * msk

    p0[...] = jnp.zeros_like(p0)
    for i in range(_BB):
        for pln in range(4):
            pp, q = pln // 2, pln % 2
            for ii in range(17):
                base = i * 1280 + (2 * ii + pp) * 36 + q
                dst = (i * 4 + pln) * _P0R + ii * 17
                p0[dst:dst + 17, :] = (
                    p0d[pl.Slice(base, 17, 2), :].astype(_BF16))

    for i in range(_BB):
        for t in range(9):
            dy, dx = t // 3, t % 3
            pln = (dy % 2) * 2 + (dx % 2)
            st = (dy // 2) * 17 + dx // 2
            src = (i * 4 + pln) * _P0R + st
            l0h[i * 272:(i + 1) * 272, t * 128:(t + 1) * 128] = (
                p0[src:src + 272, :])
    b1 = b1_ref[...]
    y1p[...] = jnp.zeros_like(y1p)
    for s in range(2):
        h = jnp.dot(l0h[s * 1088:(s + 1) * 1088], w1_ref[...],
                    preferred_element_type=jnp.float32)
        y = jnp.maximum(h[:, :128] + h[:, 128:] + b1, 0.0)
        for im in range(_BB // 2):
            i = s * 4 + im
            for yo in range(16):
                y1p[i * 360 + (yo + 1) * 18 + 1:i * 360 + (yo + 1) * 18 + 17,
                    :] = y[im * 272 + yo * 17:im * 272 + yo * 17 + 16, :]

    for i in range(_BB):
        for pln in range(4):
            pp, q = pln // 2, pln % 2
            for ii in range(10):
                base = i * 360 + (2 * ii + pp) * 18 + q
                p1[i, pln, ii * 9:ii * 9 + 9, :] = (
                    y1p[pl.Slice(base, 9, 2), :].astype(_BF16))

    for i in range(_BB):
        for t in range(9):
            dy, dx = t // 3, t % 3
            pln = (dy % 2) * 2 + (dx % 2)
            st = (dy // 2) * 9 + dx // 2
            l1h[i * 72:(i + 1) * 72, t * 128:(t + 1) * 128] = (
                p1[i, pln, st:st + 72, :])
    b2 = b2_ref[...]
    y2p[...] = jnp.zeros_like(y2p)
    for s in range(2):
        h = jnp.dot(l1h[s * 288:(s + 1) * 288], w2_ref[...],
                    preferred_element_type=jnp.float32)
        y = jnp.maximum(h + b2, 0.0).astype(_BF16)
        for im in range(_BB // 2):
            i = s * 4 + im
            for yo in range(8):
                y2p[i, (yo + 1) * 10 + 1:(yo + 1) * 10 + 9, :] = (
                    y[im * 72 + yo * 9:im * 72 + yo * 9 + 8, :])

    for i in range(_BB):
        for t in range(4):
            p, q = t // 2, t % 2
            l2h[i * 96:i * 96 + 91, t * 256:(t + 1) * 256] = (
                y2p[i, p * 10 + q:p * 10 + q + 91, :])
    b3 = b3_ref[...]
    for g in range(4):
        ga, gb = g // 2, g % 2
        h = jnp.dot(l2h[...], w3_ref[g], preferred_element_type=jnp.float32)
        y = jnp.maximum(h + b3, 0.0).astype(_BF16)
        for i in range(_BB):
            base = i * 96 + ga * 10 + gb
            blk = y[base:base + 80].reshape(8, 10, 256)[:, :8, :]
            o_ref[i, :, ga, :, gb * 256:(gb + 1) * 256] = blk


def _conv_call(a0, mask, rhs0, b0, rhs1, b1, w1, b1c, w2, b2c):
    n = a0.shape[0]
    return pl.pallas_call(
        _conv_kernel,
        out_shape=jax.ShapeDtypeStruct((n, 8, 2, 8, 512), _BF16),
        grid=(n // _BB,),
        in_specs=[
            pl.BlockSpec((_BB, 3, 1408), lambda b: (b, 0, 0)),
            pl.BlockSpec((1280, 128), lambda b: (0, 0)),
            pl.BlockSpec((32, 256), lambda b: (0, 0)),
            pl.BlockSpec((1, 128), lambda b: (0, 0)),
            pl.BlockSpec((1152, 256), lambda b: (0, 0)),
            pl.BlockSpec((1, 128), lambda b: (0, 0)),
            pl.BlockSpec((1152, 256), lambda b: (0, 0)),
            pl.BlockSpec((1, 256), lambda b: (0, 0)),
            pl.BlockSpec((4, 1024, 256), lambda b: (0, 0, 0)),
            pl.BlockSpec((1, 256), lambda b: (0, 0)),
        ],
        out_specs=pl.BlockSpec((_BB, 8, 2, 8, 512), lambda b: (b, 0, 0, 0, 0)),
        scratch_shapes=[
            pltpu.VMEM((32, 1280), _BF16),
            pltpu.VMEM((_BB * 1280, 128), jnp.float32),
            pltpu.VMEM((_BB * 4 * _P0R, 128), _BF16),
            pltpu.VMEM((_BB * 272, 1152), _BF16),
            pltpu.VMEM((_BB * 360, 128), jnp.float32),
            pltpu.VMEM((_BB, 4, 96, 128), _BF16),
            pltpu.VMEM((_BB * 72, 1152), _BF16),
            pltpu.VMEM((_BB, 110, 256), _BF16),
            pltpu.VMEM((_BB * 96, 1024), _BF16),
        ],
        compiler_params=_cparams(("parallel",)),
    )(a0, mask, rhs0, b0, rhs1, b1, w1, b1c, w2, b2c)


def _fc_kernel(a_ref, w1_ref, b1_ref, w2_ref, o_ref, acc_ref):
    i = pl.program_id(1)

    @pl.when(i == 0)
    def _():
        acc_ref[...] = jnp.zeros_like(acc_ref)

    a = a_ref[...]
    acc_ref[...] += (
        jnp.dot(a[:, :2048], w1_ref[0:2048], preferred_element_type=jnp.float32)
        + jnp.dot(a[:, 2048:], w1_ref[2048:4096],
                  preferred_element_type=jnp.float32))

    @pl.when(i == pl.num_programs(1) - 1)
    def _():
        h = jnp.maximum(acc_ref[...] + b1_ref[...], 0.0).astype(_BF16)
        o_ref[0] = jnp.dot(h, w2_ref[...], preferred_element_type=jnp.float32)


def _fc_call(a, w1, b1, w2):
    mp, k = a.shape
    tk = 4096
    return pl.pallas_call(
        _fc_kernel,
        out_shape=jax.ShapeDtypeStruct((2, mp, 128), jnp.float32),
        grid=(2, k // tk),
        in_specs=[
            pl.BlockSpec((mp, tk), lambda j, i: (0, i)),
            pl.BlockSpec((tk, 256), lambda j, i: (i, j)),
            pl.BlockSpec((1, 256), lambda j, i: (0, j)),
            pl.BlockSpec((256, 128), lambda j, i: (j, 0)),
        ],
        out_specs=pl.BlockSpec((1, mp, 128), lambda j, i: (j, 0, 0)),
        scratch_shapes=[pltpu.VMEM((mp, 256), jnp.float32)],
        compiler_params=_cparams(("parallel", "arbitrary")),
    )(a, w1, b1, w2)


def kernel(x, init_w, init_b, layer0_w, layer0_b, layer1_w, layer1_b,
           layer2_w, layer2_b, fc1_w, fc1_b, fc2_w, fc2_b):
    n = x.shape[0]

    xb = jnp.pad(x.astype(_BF16), ((0, 0), (0, 0), (2, 2), (2, 2)))
    a0 = jnp.pad(xb.reshape(n, 3, 1296), ((0, 0), (0, 0), (0, 112)))

    r = jnp.arange(1280)
    u, v = r // 36, r % 36
    ok = (u >= 1) & (u <= 32) & (v >= 1) & (v <= 32)
    mask = jnp.broadcast_to(ok[:, None], (1280, 128)).astype(jnp.float32)

    w27 = init_w.reshape(9, 128, 128)[:, :3, :].reshape(27, 128)
    k27 = jnp.arange(27)[:, None]
    rhs0 = jnp.concatenate(
        [jnp.where(k27 < 15, w27, 0), jnp.where(k27 >= 15, w27, 0)], axis=1)
    rhs0 = jnp.pad(rhs0, ((0, 5), (0, 0)))

    w0 = layer0_w[0]
    kk = jnp.arange(1152)[:, None]
    rhs1 = jnp.concatenate(
        [jnp.where(kk < 640, w0, 0), jnp.where(kk >= 640, w0, 0)], axis=1)

    y3 = _conv_call(a0, mask, rhs0, init_b, rhs1, layer0_b,
                    layer1_w[0], layer1_b, layer2_w, layer2_b)

    a = y3.reshape(n, 65536)
    parts = _fc_call(a, fc1_w, fc1_b, fc2_w)
    out = parts[0] + parts[1] + fc2_b
    return out[:, :10]


# speedup vs baseline: 3.3750x; 1.0471x over previous
import jax
import jax.numpy as jnp
from jax.experimental import pallas as pl
from jax.experimental.pallas import tpu as pltpu

_BF16 = jnp.bfloat16
_VLIM = int(56 * 2**20)
_BB = 8
_P0R = 312


def _cparams(sem):
    return pltpu.CompilerParams(dimension_semantics=sem, vmem_limit_bytes=_VLIM)


def _conv_kernel(a_ref, m_ref, w0_ref, b0_ref, w1_ref, b1_ref, w2_ref, b2_ref,
                 w3_ref, b3_ref, o_ref, lhsT, p0d, p0, l0h, y1p, p1, l1h, y2p,
                 l2h):
    b0 = b0_ref[...]
    msk = m_ref[...]
    lhsT[27:32, :] = jnp.zeros((5, 1280), _BF16)
    for i in range(_BB):
        for t in range(9):
            dy, dx = t // 3, t % 3
            sh = dy * 36 + dx
            lhsT[t * 3:(t + 1) * 3, :] = a_ref[i, :, sh:sh + 1280]
        h = jnp.dot(lhsT[...].T, w0_ref[...],
                    preferred_element_type=jnp.float32)
        y = jnp.maximum(h[:, :128] + h[:, 128:] + b0, 0.0)
        p0d[i * 1280:(i + 1) * 1280, :] = y * msk

    p0[...] = jnp.zeros_like(p0)
    for i in range(_BB):
        for pln in range(4):
            pp, q = pln // 2, pln % 2
            for ii in range(17):
                base = i * 1280 + (2 * ii + pp) * 36 + q
                dst = (i * 4 + pln) * _P0R + ii * 17
                p0[dst:dst + 17, :] = (
                    p0d[pl.Slice(base, 17, 2), :].astype(_BF16))

    for i in range(_BB):
        for t in range(9):
            dy, dx = t // 3, t % 3
            pln = (dy % 2) * 2 + (dx % 2)
            st = (dy // 2) * 17 + dx // 2
            src = (i * 4 + pln) * _P0R + st
            l0h[i * 272:(i + 1) * 272, t * 128:(t + 1) * 128] = (
                p0[src:src + 272, :])
    b1 = b1_ref[...]
    y1p[...] = jnp.zeros_like(y1p)
    for s in range(2):
        h = jnp.dot(l0h[s * 1088:(s + 1) * 1088], w1_ref[...],
                    preferred_element_type=jnp.float32)
        y = jnp.maximum(h[:, :128] + h[:, 128:] + b1, 0.0)
        for im in range(_BB // 2):
            i = s * 4 + im
            for yo in range(16):
                y1p[i * 360 + (yo + 1) * 18 + 1:i * 360 + (yo + 1) * 18 + 17,
                    :] = y[im * 272 + yo * 17:im * 272 + yo * 17 + 16, :]

    for i in range(_BB):
        for pln in range(4):
            pp, q = pln // 2, pln % 2
            for ii in range(10):
                base = i * 360 + (2 * ii + pp) * 18 + q
                p1[i, pln, ii * 9:ii * 9 + 9, :] = (
                    y1p[pl.Slice(base, 9, 2), :].astype(_BF16))

    for i in range(_BB):
        for t in range(9):
            dy, dx = t // 3, t % 3
            pln = (dy % 2) * 2 + (dx % 2)
            st = (dy // 2) * 9 + dx // 2
            l1h[i * 72:(i + 1) * 72, t * 128:(t + 1) * 128] = (
                p1[i, pln, st:st + 72, :])
    b2 = b2_ref[...]
    y2p[...] = jnp.zeros_like(y2p)
    for s in range(2):
        h = jnp.dot(l1h[s * 288:(s + 1) * 288], w2_ref[...],
                    preferred_element_type=jnp.float32)
        y = jnp.maximum(h + b2, 0.0).astype(_BF16)
        for im in range(_BB // 2):
            i = s * 4 + im
            for yo in range(8):
                y2p[i, (yo + 1) * 10 + 1:(yo + 1) * 10 + 9, :] = (
                    y[im * 72 + yo * 9:im * 72 + yo * 9 + 8, :])

    for i in range(_BB):
        for t in range(4):
            p, q = t // 2, t % 2
            l2h[i * 96:i * 96 + 91, t * 256:(t + 1) * 256] = (
                y2p[i, p * 10 + q:p * 10 + q + 91, :])
    b3 = b3_ref[...]
    for g in range(4):
        ga, gb = g // 2, g % 2
        h = jnp.dot(l2h[...], w3_ref[g], preferred_element_type=jnp.float32)
        y = jnp.maximum(h + b3, 0.0).astype(_BF16)
        for i in range(_BB):
            base = i * 96 + ga * 10 + gb
            blk = y[base:base + 80].reshape(8, 10, 256)[:, :8, :]
            o_ref[i, :, ga, :, gb * 256:(gb + 1) * 256] = blk


def _conv_call(a0, mask, rhs0, b0, rhs1, b1, w1, b1c, w2, b2c):
    n = a0.shape[0]
    return pl.pallas_call(
        _conv_kernel,
        out_shape=jax.ShapeDtypeStruct((n, 8, 2, 8, 512), _BF16),
        grid=(n // _BB,),
        in_specs=[
            pl.BlockSpec((_BB, 3, 1408), lambda b: (b, 0, 0)),
            pl.BlockSpec((1280, 128), lambda b: (0, 0)),
            pl.BlockSpec((32, 256), lambda b: (0, 0)),
            pl.BlockSpec((1, 128), lambda b: (0, 0)),
            pl.BlockSpec((1152, 256), lambda b: (0, 0)),
            pl.BlockSpec((1, 128), lambda b: (0, 0)),
            pl.BlockSpec((1152, 256), lambda b: (0, 0)),
            pl.BlockSpec((1, 256), lambda b: (0, 0)),
            pl.BlockSpec((4, 1024, 256), lambda b: (0, 0, 0)),
            pl.BlockSpec((1, 256), lambda b: (0, 0)),
        ],
        out_specs=pl.BlockSpec((_BB, 8, 2, 8, 512), lambda b: (b, 0, 0, 0, 0)),
        scratch_shapes=[
            pltpu.VMEM((32, 1280), _BF16),
            pltpu.VMEM((_BB * 1280, 128), jnp.float32),
            pltpu.VMEM((_BB * 4 * _P0R, 128), _BF16),
            pltpu.VMEM((_BB * 272, 1152), _BF16),
            pltpu.VMEM((_BB * 360, 128), jnp.float32),
            pltpu.VMEM((_BB, 4, 96, 128), _BF16),
            pltpu.VMEM((_BB * 72, 1152), _BF16),
            pltpu.VMEM((_BB, 110, 256), _BF16),
            pltpu.VMEM((_BB * 96, 1024), _BF16),
        ],
        compiler_params=_cparams(("parallel",)),
    )(a0, mask, rhs0, b0, rhs1, b1, w1, b1c, w2, b2c)


def _fc_kernel(a_ref, w1_ref, b1_ref, w2_ref, o_ref, acc_ref):
    i = pl.program_id(1)

    @pl.when(i == 0)
    def _():
        acc_ref[...] = jnp.zeros_like(acc_ref)

    upd = jnp.zeros_like(acc_ref)
    for s in range(8):
        upd += jnp.dot(a_ref[:, 0, 0, s, :], w1_ref[s * 512:(s + 1) * 512],
                       preferred_element_type=jnp.float32)
    acc_ref[...] += upd

    @pl.when(i == pl.num_programs(1) - 1)
    def _():
        h = jnp.maximum(acc_ref[...] + b1_ref[...], 0.0).astype(_BF16)
        o_ref[0] = jnp.dot(h, w2_ref[...], preferred_element_type=jnp.float32)


def _fc_call(a, w1, b1, w2):
    mp = a.shape[0]
    tk = 4096
    return pl.pallas_call(
        _fc_kernel,
        out_shape=jax.ShapeDtypeStruct((2, mp, 128), jnp.float32),
        grid=(2, 16),
        in_specs=[
            pl.BlockSpec((mp, 1, 1, 8, 512),
                         lambda j, i: (0, i // 2, i % 2, 0, 0)),
            pl.BlockSpec((tk, 256), lambda j, i: (i, j)),
            pl.BlockSpec((1, 256), lambda j, i: (0, j)),
            pl.BlockSpec((256, 128), lambda j, i: (j, 0)),
        ],
        out_specs=pl.BlockSpec((1, mp, 128), lambda j, i: (j, 0, 0)),
        scratch_shapes=[pltpu.VMEM((mp, 256), jnp.float32)],
        compiler_params=_cparams(("parallel", "arbitrary")),
    )(a, w1, b1, w2)


def kernel(x, init_w, init_b, layer0_w, layer0_b, layer1_w, layer1_b,
           layer2_w, layer2_b, fc1_w, fc1_b, fc2_w, fc2_b):
    n = x.shape[0]

    xb = jnp.pad(x.astype(_BF16), ((0, 0), (0, 0), (2, 2), (2, 2)))
    a0 = jnp.pad(xb.reshape(n, 3, 1296), ((0, 0), (0, 0), (0, 112)))

    r = jnp.arange(1280)
    u, v = r // 36, r % 36
    ok = (u >= 1) & (u <= 32) & (v >= 1) & (v <= 32)
    mask = jnp.broadcast_to(ok[:, None], (1280, 128)).astype(jnp.float32)

    w27 = init_w.reshape(9, 128, 128)[:, :3, :].reshape(27, 128)
    k27 = jnp.arange(27)[:, None]
    rhs0 = jnp.concatenate(
        [jnp.where(k27 < 15, w27, 0), jnp.where(k27 >= 15, w27, 0)], axis=1)
    rhs0 = jnp.pad(rhs0, ((0, 5), (0, 0)))

    w0 = layer0_w[0]
    kk = jnp.arange(1152)[:, None]
    rhs1 = jnp.concatenate(
        [jnp.where(kk < 640, w0, 0), jnp.where(kk >= 640, w0, 0)], axis=1)

    y3 = _conv_call(a0, mask, rhs0, init_b, rhs1, layer0_b,
                    layer1_w[0], layer1_b, layer2_w, layer2_b)

    parts = _fc_call(y3, fc1_w, fc1_b, fc2_w)
    out = parts[0] + parts[1] + fc2_b
    return out[:, :10]
